# baseline (speedup 1.0000x reference)
import functools

import jax
import jax.numpy as jnp
from jax import lax
from jax.experimental import pallas as pl
from jax.experimental.pallas import tpu as pltpu
from jax.experimental.pallas import tpu_sc as plsc

NUM_CORES = 2
NUM_SUBCORES = 16
NW = NUM_CORES * NUM_SUBCORES
GRP = 4


@functools.partial(jax.jit, static_argnames=("batch", "seq", "emb"))
def _sc_gather(W, x, *, batch, seq, emb):
    rows_per_w = batch // NW
    n_groups = rows_per_w // GRP
    nbuf = 4
    outer = n_groups // nbuf
    mesh = plsc.VectorSubcoreMesh(core_axis_name="c", subcore_axis_name="s")

    @functools.partial(
        pl.kernel,
        mesh=mesh,
        out_type=jax.ShapeDtypeStruct((batch, seq, emb), jnp.float32),
        scratch_types=[
            pltpu.VMEM((rows_per_w, 64), jnp.float32),
            pltpu.VMEM((rows_per_w, 64), jnp.int32),
            pltpu.VMEM((nbuf, GRP, 64, emb), jnp.float32),
            pltpu.SemaphoreType.DMA((nbuf,)),
            pltpu.SemaphoreType.DMA((nbuf,)),
        ],
        compiler_params=pltpu.CompilerParams(use_tc_tiling_on_sc=False),
    )
    def k(W_hbm, x_hbm, out_hbm, idx_f, idx_v, rows_v, gsem, wsem):
        wid = lax.axis_index("s") * NUM_CORES + lax.axis_index("c")
        base = wid * rows_per_w
        pltpu.sync_copy(x_hbm.at[pl.ds(base, rows_per_w)], idx_f)

        def cvt(j, carry):
            for c in range(4):
                idx_v[j, pl.ds(c * 16, 16)] = idx_f[j, pl.ds(c * 16, 16)].astype(
                    jnp.int32
                )
            return carry

        lax.fori_loop(0, rows_per_w, cvt, 0)

        def gather_desc(g, b, i):
            idx = idx_v.at[g * GRP + i]
            return pltpu.make_async_copy(W_hbm.at[idx], rows_v.at[b, i], gsem.at[b])

        def wb_desc(g, b):
            dst = out_hbm.at[pl.ds(base + g * GRP, GRP)]
            src = rows_v.at[b, :, pl.ds(0, seq)]
            return pltpu.make_async_copy(src, dst, wsem.at[b])

        def start_group(g, b):
            for i in range(GRP):
                gather_desc(g, b, i).start()

        def wait_group(g, b):
            for i in range(GRP):
                gather_desc(g, b, i).wait()

        for b in range(nbuf):
            start_group(b, b)

        def body(t, carry):
            g0 = t * nbuf
            for b in range(nbuf):
                wait_group(g0 + b, b)
                wb_desc(g0 + b, b).start()

            @pl.when(t < outer - 1)
            def _rearm():
                for b in range(nbuf):
                    wb_desc(g0 + b, b).wait()
                    start_group(g0 + nbuf + b, b)

            return carry

        lax.fori_loop(0, outer, body, 0)
        glast = (outer - 1) * nbuf
        for b in range(nbuf):
            wb_desc(glast + b, b).wait()

    return k(W, x)


def kernel(x, W):
    b, s = x.shape
    xpf = jnp.pad(x.astype(jnp.float32), ((0, 0), (0, 64 - s)))
    return _sc_gather(W, xpf, batch=b, seq=s, emb=W.shape[1])


# speedup vs baseline: 5.7233x; 5.7233x over previous
import functools

import jax
import jax.numpy as jnp
from jax import lax
from jax.experimental import pallas as pl
from jax.experimental.pallas import tpu as pltpu
from jax.experimental.pallas import tpu_sc as plsc

NUM_CORES = 2
NUM_SUBCORES = 16
NW = NUM_CORES * NUM_SUBCORES
CHUNK = 128


@functools.partial(jax.jit, static_argnames=("n_rows", "emb"))
def _sc_gather(W, xf, *, n_rows, emb):
    rows_per_w = n_rows // NW
    chunks_per_w = rows_per_w // CHUNK
    nbuf = 5
    outer = chunks_per_w // nbuf
    mesh = plsc.VectorSubcoreMesh(core_axis_name="c", subcore_axis_name="s")

    @functools.partial(
        pl.kernel,
        mesh=mesh,
        out_type=jax.ShapeDtypeStruct((n_rows, emb), jnp.float32),
        scratch_types=[
            pltpu.VMEM((rows_per_w,), jnp.int32),
            pltpu.VMEM((nbuf, CHUNK, emb), jnp.float32),
            pltpu.SemaphoreType.DMA((nbuf,)),
            pltpu.SemaphoreType.DMA((nbuf,)),
        ],
        compiler_params=pltpu.CompilerParams(use_tc_tiling_on_sc=False),
    )
    def k(W_hbm, xf_hbm, out_hbm, idx_v, rows_v, gsem, wsem):
        wid = lax.axis_index("s") * NUM_CORES + lax.axis_index("c")
        base = wid * rows_per_w
        pltpu.sync_copy(xf_hbm.at[pl.ds(base, rows_per_w)], idx_v)

        def gather_desc(j, b):
            idx = idx_v.at[pl.ds(j * CHUNK, CHUNK)]
            return pltpu.make_async_copy(W_hbm.at[idx], rows_v.at[b], gsem.at[b])

        def wb_desc(j, b):
            dst = out_hbm.at[pl.ds(base + j * CHUNK, CHUNK)]
            return pltpu.make_async_copy(rows_v.at[b], dst, wsem.at[b])

        for b in range(nbuf):
            gather_desc(b, b).start()

        def body(t, carry):
            j0 = t * nbuf
            for b in range(nbuf):
                gather_desc(j0 + b, b).wait()
                wb_desc(j0 + b, b).start()

            @pl.when(t < outer - 1)
            def _rearm():
                for b in range(nbuf):
                    wb_desc(j0 + b, b).wait()
                    gather_desc(j0 + nbuf + b, b).start()

            return carry

        lax.fori_loop(0, outer, body, 0)
        jlast = (outer - 1) * nbuf
        for b in range(nbuf):
            wb_desc(jlast + b, b).wait()

    return k(W, xf)


def kernel(x, W):
    b, s = x.shape
    n_rows = b * s
    xf = x.reshape(n_rows).astype(jnp.int32)
    out = _sc_gather(W, xf, n_rows=n_rows, emb=W.shape[1])
    return out.reshape(b, s, W.shape[1])


# speedup vs baseline: 5.7391x; 1.0027x over previous
import functools

import jax
import jax.numpy as jnp
from jax import lax
from jax.experimental import pallas as pl
from jax.experimental.pallas import tpu as pltpu
from jax.experimental.pallas import tpu_sc as plsc

NUM_CORES = 2
NUM_SUBCORES = 16
NW = NUM_CORES * NUM_SUBCORES
CHUNK = 128


@functools.partial(jax.jit, static_argnames=("n_rows", "emb"))
def _sc_gather(W, xf, *, n_rows, emb):
    rows_per_w = n_rows // NW
    chunks_per_w = rows_per_w // CHUNK
    nbuf = 10
    outer = chunks_per_w // nbuf
    mesh = plsc.VectorSubcoreMesh(core_axis_name="c", subcore_axis_name="s")

    @functools.partial(
        pl.kernel,
        mesh=mesh,
        out_type=jax.ShapeDtypeStruct((n_rows, emb), jnp.float32),
        scratch_types=[
            pltpu.VMEM((rows_per_w,), jnp.int32),
            pltpu.VMEM((nbuf, CHUNK, emb), jnp.float32),
            pltpu.SemaphoreType.DMA((nbuf,)),
            pltpu.SemaphoreType.DMA((nbuf,)),
        ],
        compiler_params=pltpu.CompilerParams(use_tc_tiling_on_sc=False),
    )
    def k(W_hbm, xf_hbm, out_hbm, idx_v, rows_v, gsem, wsem):
        wid = lax.axis_index("s") * NUM_CORES + lax.axis_index("c")
        base = wid * rows_per_w
        pltpu.sync_copy(xf_hbm.at[pl.ds(base, rows_per_w)], idx_v)

        def gather_desc(j, b):
            idx = idx_v.at[pl.ds(j * CHUNK, CHUNK)]
            return pltpu.make_async_copy(W_hbm.at[idx], rows_v.at[b], gsem.at[b])

        def wb_desc(j, b):
            dst = out_hbm.at[pl.ds(base + j * CHUNK, CHUNK)]
            return pltpu.make_async_copy(rows_v.at[b], dst, wsem.at[b])

        for b in range(nbuf):
            gather_desc(b, b).start()

        def body(t, carry):
            j0 = t * nbuf
            for b in range(nbuf):
                gather_desc(j0 + b, b).wait()
                wb_desc(j0 + b, b).start()

            @pl.when(t < outer - 1)
            def _rearm():
                for b in range(nbuf):
                    wb_desc(j0 + b, b).wait()
                    gather_desc(j0 + nbuf + b, b).start()

            return carry

        lax.fori_loop(0, outer, body, 0)
        jlast = (outer - 1) * nbuf
        for b in range(nbuf):
            wb_desc(jlast + b, b).wait()

    return k(W, xf)


def kernel(x, W):
    b, s = x.shape
    n_rows = b * s
    xf = x.reshape(n_rows).astype(jnp.int32)
    out = _sc_gather(W, xf, n_rows=n_rows, emb=W.shape[1])
    return out.reshape(b, s, W.shape[1])
